# Initial kernel scaffold; baseline (speedup 1.0000x reference)
import jax
import jax.numpy as jnp
from jax.experimental import pallas as pl


def kernel(x, edge_index):
    raise NotImplementedError("write your pallas kernel here")


# speedup vs baseline: 5.5570x; 5.5570x over previous
import functools

import jax
import jax.numpy as jnp
from jax import lax
from jax.experimental import pallas as pl
from jax.experimental.pallas import tpu as pltpu
from jax.experimental.pallas import tpu_sc as plsc

NC = 2
NS = 16
LANES = 16
CHUNK = 80
ZROWS = 80


def _sc_partial_sums(x, src, dst):
    n, d = x.shape
    e = src.shape[0]
    n_blocks = n // ZROWS
    blocks_per_tile = -(-n_blocks // NS)
    edges_per_core = e // NC
    edges_per_tile = edges_per_core // NS

    mesh = plsc.VectorSubcoreMesh(core_axis_name="c", subcore_axis_name="s")

    @functools.partial(
        pl.kernel,
        out_type=(jax.ShapeDtypeStruct((n, d), jnp.float32),
                  jax.ShapeDtypeStruct((n, d), jnp.float32)),
        mesh=mesh,
        scratch_types=[
            pltpu.VMEM((CHUNK,), jnp.int32),
            pltpu.VMEM((CHUNK,), jnp.int32),
            pltpu.VMEM((CHUNK, d), jnp.float32),
            pltpu.VMEM((ZROWS, d), jnp.float32),
            pltpu.VMEM_SHARED((n, d), jnp.float32),
            pltpu.SemaphoreType.DMA,
        ],
    )
    def sc_kern(x_hbm, src_hbm, dst_hbm, p0_hbm, p1_hbm,
                sidx, didx, rows, zbuf, acc, sem):
        c = lax.axis_index("c")
        s = lax.axis_index("s")

        @pl.loop(0, ZROWS)
        def _(r):
            @pl.loop(0, d, step=LANES)
            def _(cc):
                zbuf[r, pl.ds(cc, LANES)] = jnp.zeros((LANES,), jnp.float32)

        @pl.loop(0, blocks_per_tile)
        def _(i):
            blk = i * NS + s

            @pl.when(blk < n_blocks)
            def _():
                pltpu.sync_copy(zbuf, acc.at[pl.ds(blk * ZROWS, ZROWS)])

        plsc.subcore_barrier()

        @pl.loop(0, edges_per_tile, step=CHUNK)
        def _(e0):
            base = c * edges_per_core + s * edges_per_tile + e0
            pltpu.sync_copy(src_hbm.at[pl.ds(base, CHUNK)], sidx)
            pltpu.sync_copy(dst_hbm.at[pl.ds(base, CHUNK)], didx)
            pltpu.async_copy(x_hbm.at[sidx], rows, sem).wait()
            pltpu.sync_copy(rows, acc.at[didx], add=True)

        plsc.subcore_barrier()

        @pl.loop(0, blocks_per_tile)
        def _(i):
            blk = i * NS + s

            @pl.when(blk < n_blocks)
            def _():
                r = blk * ZROWS
                pltpu.sync_copy(acc.at[pl.ds(r, ZROWS)], zbuf)

                @pl.when(c == 0)
                def _():
                    pltpu.sync_copy(zbuf, p0_hbm.at[pl.ds(r, ZROWS)])

                @pl.when(c == 1)
                def _():
                    pltpu.sync_copy(zbuf, p1_hbm.at[pl.ds(r, ZROWS)])

    return sc_kern(x, src, dst)


def _tc_add(a, b):
    n, d = a.shape
    bt = 1000

    def body(a_ref, b_ref, o_ref):
        o_ref[...] = a_ref[...] + b_ref[...]

    return pl.pallas_call(
        body,
        out_shape=jax.ShapeDtypeStruct((n, d), jnp.float32),
        grid=(n // bt,),
        in_specs=[pl.BlockSpec((bt, d), lambda i: (i, 0)),
                  pl.BlockSpec((bt, d), lambda i: (i, 0))],
        out_specs=pl.BlockSpec((bt, d), lambda i: (i, 0)),
    )(a, b)


def kernel(x, edge_index):
    src = edge_index[0]
    dst = edge_index[1]
    p0, p1 = _sc_partial_sums(x, src, dst)
    return _tc_add(p0, p1)
